# baseline (speedup 1.0000x reference)
import functools

import jax
import jax.numpy as jnp
from jax import lax
from jax.experimental import pallas as pl
from jax.experimental.pallas import tpu as pltpu
from jax.experimental.pallas import tpu_sc as plsc

_F32 = jnp.float32


def _pick_k(n_chunks):
  for k in (10, 8, 5, 4, 2, 1):
    if n_chunks % k == 0 and n_chunks // k >= 32:
      return k
  for k in (10, 8, 5, 4, 2, 1):
    if n_chunks % k == 0:
      return k
  return 1


def _sc_gather(table, idx_flat):
  v, d = table.shape
  dt = table.dtype
  b = idx_flat.shape[0]
  n_chunks = b // 128
  k = _pick_k(n_chunks)
  c = k * 128
  n_sup = n_chunks // k
  n_loop = -(-n_sup // 32)
  idx3d = idx_flat.reshape(n_sup, k, 128)
  mesh = plsc.VectorSubcoreMesh(core_axis_name="c", subcore_axis_name="s")

  @functools.partial(
      pl.kernel,
      mesh=mesh,
      compiler_params=pltpu.CompilerParams(use_tc_tiling_on_sc=False),
      out_type=jax.ShapeDtypeStruct((b, d), dt),
      scratch_types=[
          pltpu.VMEM((k, 128), jnp.int32),
          pltpu.VMEM((c, d), dt),
          pltpu.SemaphoreType.DMA,
      ],
  )
  def gather(table_hbm, idx_hbm, out_hbm, idx_v, rows_v, sem):
    wid = lax.axis_index("s") * 2 + lax.axis_index("c")

    def body(s, carry):
      sup = s * 32 + wid

      @pl.when(sup < n_sup)
      def _():
        pltpu.sync_copy(idx_hbm.at[sup], idx_v)
        copies = [
            pltpu.async_copy(
                table_hbm.at[idx_v.at[j]],
                rows_v.at[pl.ds(j * 128, 128)],
                sem,
            )
            for j in range(k)
        ]
        for cp in copies:
          cp.wait()
        pltpu.sync_copy(rows_v, out_hbm.at[pl.ds(sup * c, c)])

      return carry

    lax.fori_loop(0, n_loop, body, 0)

  return gather(table, idx3d)


def _pick_block(n, cap):
  for r in range(min(n, cap), 0, -1):
    if n % r == 0 and (r % 8 == 0 or r == n):
      return r
  return n


def _embed_body(a_ref, m_ref, w_ref, masked_ref, af_ref):
  masked = a_ref[...] * m_ref[...]
  masked_ref[...] = masked
  af_ref[...] = jnp.dot(masked, w_ref[...], preferred_element_type=_F32)


def _embed(atom_fea, mask2d, w_embt):
  n, orig = atom_fea.shape
  af_dim = w_embt.shape[1]
  r = _pick_block(n, 2000)
  grid = (n // r,)
  return pl.pallas_call(
      _embed_body,
      grid=grid,
      in_specs=[
          pl.BlockSpec((r, orig), lambda i: (i, 0)),
          pl.BlockSpec((1, orig), lambda i: (0, 0)),
          pl.BlockSpec((orig, af_dim), lambda i: (0, 0)),
      ],
      out_specs=[
          pl.BlockSpec((r, orig), lambda i: (i, 0)),
          pl.BlockSpec((r, af_dim), lambda i: (i, 0)),
      ],
      out_shape=[
          jax.ShapeDtypeStruct((n, orig), _F32),
          jax.ShapeDtypeStruct((n, af_dim), _F32),
      ],
  )(atom_fea, mask2d, w_embt)


def _edge_proj(gpk_ref, nbrp_ref, af_ref, wbd, w8, wsp, bp):
  ep = jnp.dot(gpk_ref[...], wbd, preferred_element_type=_F32)
  nq = jnp.dot(nbrp_ref[...], w8, preferred_element_type=_F32)
  parts = [nq[:, 128 * j:128 * (j + 1)] for j in range(4)]
  ep = ep + jnp.concatenate(parts, axis=0)
  sp = jnp.dot(af_ref[...], wsp, preferred_element_type=_F32)
  sp = sp + bp
  return ep, sp


def _stats_half(r, ppa, e, s):
  esum = jnp.sum(e.reshape(r, ppa, 128), axis=1)
  tot = jnp.sum(esum, axis=0) + ppa * jnp.sum(s, axis=0)
  tot2 = (jnp.sum(e * e, axis=0) + 2.0 * jnp.sum(s * esum, axis=0)
          + ppa * jnp.sum(s * s, axis=0))
  return tot, tot2


def _stats_body(m, gpk_ref, nbrp_ref, af_ref, wbdf, w8f, wspf, bpf, wbdc, w8c,
                wspc, bpc, out_ref):
  r = af_ref.shape[0]
  ppa = m // 2
  ef, sf = _edge_proj(gpk_ref, nbrp_ref, af_ref, wbdf[...], w8f[...],
                      wspf[...], bpf[...])
  ec, sc2 = _edge_proj(gpk_ref, nbrp_ref, af_ref, wbdc[...], w8c[...],
                       wspc[...], bpc[...])
  tf, tf2 = _stats_half(r, ppa, ef, sf)
  tc, tc2 = _stats_half(r, ppa, ec, sc2)

  @pl.when(pl.program_id(0) == 0)
  def _():
    out_ref[...] = jnp.zeros_like(out_ref)

  out_ref[0:1, :] += tf[None, :]
  out_ref[1:2, :] += tf2[None, :]
  out_ref[2:3, :] += tc[None, :]
  out_ref[3:4, :] += tc2[None, :]


def _main_body(m, cnt, gpk_ref, nbrp_ref, af_ref, wbdf, w8f, wspf, bpf, wbdc,
               w8c, wspc, bpc, st_ref, stb_ref, g1f, b1f, g1c, b1c, p_ref,
               pt_ref, sum_ref, out2_ref):
  r = af_ref.shape[0]
  pairs_per_atom = m // 2
  inv = 1.0 / cnt
  p = p_ref[...]
  pt = pt_ref[...]
  st = jnp.dot(st_ref[...] + stb_ref[...], p,
               preferred_element_type=_F32)
  mf = st[0:1, :] * inv
  vf = st[1:2, :] * inv - mf * mf
  scf = g1f[...] * lax.rsqrt(vf + 1e-5)
  shf = b1f[...] - mf * scf
  mc = st[2:3, :] * inv
  vc = st[3:4, :] * inv - mc * mc
  scc = g1c[...] * lax.rsqrt(vc + 1e-5)
  shc = b1c[...] - mc * scc
  scf_p = jnp.dot(scf, pt, preferred_element_type=_F32)
  shf_p = jnp.dot(shf, pt, preferred_element_type=_F32)
  scc_p = jnp.dot(scc, pt, preferred_element_type=_F32)
  shc_p = jnp.dot(shc, pt, preferred_element_type=_F32)

  scf_p4 = jnp.concatenate([scf_p] * 4, axis=1)
  scc_p4 = jnp.concatenate([scc_p] * 4, axis=1)
  ef, sf = _edge_proj(gpk_ref, nbrp_ref, af_ref, wbdf[...] * scf_p,
                      w8f[...] * scf_p4, wspf[...] * scf_p,
                      bpf[...] * scf_p + shf_p)
  ec, sc2 = _edge_proj(gpk_ref, nbrp_ref, af_ref, wbdc[...] * scc_p,
                       w8c[...] * scc_p4, wspc[...] * scc_p,
                       bpc[...] * scc_p + shc_p)

  filt = jax.nn.sigmoid(ef.reshape(r, pairs_per_atom, 128) + sf[:, None, :])
  core = jax.nn.softplus(ec.reshape(r, pairs_per_atom, 128) + sc2[:, None, :])
  psum = jnp.sum(filt * core, axis=1)
  sm = jnp.dot(psum, p, preferred_element_type=_F32)
  sum_ref[...] = sm

  @pl.when(pl.program_id(0) == 0)
  def _():
    out2_ref[...] = jnp.zeros_like(out2_ref)

  out2_ref[0:1, :] += jnp.sum(sm, axis=0)[None, :]
  out2_ref[1:2, :] += jnp.sum(sm * sm, axis=0)[None, :]


def _update_body(cnt, af_ref, sm_ref, st_ref, stb_ref, g2, b2, out_ref):
  inv = 1.0 / cnt
  st = st_ref[...] + stb_ref[...]
  mu = st[0:1, :] * inv
  var = st[1:2, :] * inv - mu * mu
  s = g2[...] * lax.rsqrt(var + 1e-5)
  sh = b2[...] - mu * s
  out_ref[...] = jax.nn.softplus(af_ref[...] + sm_ref[...] * s + sh)


def _conv_layer(af_halves, gpk_halves, nbrp_halves, n_tot, m, r, wbdf, w8f,
                wspf, bpf, wbdc, w8c, wspc, bpc, g1f, b1f, g1c, b1c, g2, b2,
                p_mat, pt_mat):
  af_dim = af_halves[0].shape[1]
  nh = af_halves[0].shape[0]
  grid = (nh // r,)
  pack_r = r * m // 2
  nbr_r = nbrp_halves[0].shape[0] // (nh // r)

  g_spec = pl.BlockSpec((pack_r, 128), lambda i: (i, 0))
  nbr_spec = pl.BlockSpec((nbr_r, 128), lambda i: (i, 0))
  af_spec = pl.BlockSpec((r, af_dim), lambda i: (i, 0))
  wbd_spec = pl.BlockSpec((128, 128), lambda i: (0, 0))
  w8_spec = pl.BlockSpec((128, 512), lambda i: (0, 0))
  wsp_spec = pl.BlockSpec((af_dim, 128), lambda i: (0, 0))
  bp_spec = pl.BlockSpec((1, 128), lambda i: (0, 0))
  b_spec = pl.BlockSpec((1, af_dim), lambda i: (0, 0))
  stp_spec = pl.BlockSpec((8, 128), lambda i: (0, 0))
  st_spec = pl.BlockSpec((8, af_dim), lambda i: (0, 0))
  p_spec = pl.BlockSpec((128, af_dim), lambda i: (0, 0))
  pt_spec = pl.BlockSpec((af_dim, 128), lambda i: (0, 0))

  sts = [
      pl.pallas_call(
          functools.partial(_stats_body, m),
          grid=grid,
          in_specs=[g_spec, nbr_spec, af_spec, wbd_spec, w8_spec, wsp_spec,
                    bp_spec, wbd_spec, w8_spec, wsp_spec, bp_spec],
          out_specs=stp_spec,
          out_shape=jax.ShapeDtypeStruct((8, 128), _F32),
      )(gpk_halves[h], nbrp_halves[h], af_halves[h], wbdf, w8f, wspf, bpf,
        wbdc, w8c, wspc, bpc)
      for h in (0, 1)
  ]

  mains = [
      pl.pallas_call(
          functools.partial(_main_body, m, float(n_tot * m)),
          grid=grid,
          in_specs=[g_spec, nbr_spec, af_spec, wbd_spec, w8_spec, wsp_spec,
                    bp_spec, wbd_spec, w8_spec, wsp_spec, bp_spec, stp_spec,
                    stp_spec, b_spec, b_spec, b_spec, b_spec, p_spec,
                    pt_spec],
          out_specs=[af_spec, st_spec],
          out_shape=[
              jax.ShapeDtypeStruct((nh, af_dim), _F32),
              jax.ShapeDtypeStruct((8, af_dim), _F32),
          ],
      )(gpk_halves[h], nbrp_halves[h], af_halves[h], wbdf, w8f, wspf, bpf,
        wbdc, w8c, wspc, bpc, sts[0], sts[1], g1f, b1f, g1c, b1c, p_mat,
        pt_mat)
      for h in (0, 1)
  ]

  r2 = _pick_block(nh, 2000)
  outs = [
      pl.pallas_call(
          functools.partial(_update_body, float(n_tot)),
          grid=(nh // r2,),
          in_specs=[
              pl.BlockSpec((r2, af_dim), lambda i: (i, 0)),
              pl.BlockSpec((r2, af_dim), lambda i: (i, 0)),
              st_spec,
              st_spec,
              b_spec,
              b_spec,
          ],
          out_specs=pl.BlockSpec((r2, af_dim), lambda i: (i, 0)),
          out_shape=jax.ShapeDtypeStruct((nh, af_dim), _F32),
      )(af_halves[h], mains[h][0], mains[0][1], mains[1][1], g2, b2)
      for h in (0, 1)
  ]
  return outs


def _readout_body(ncry, apc, g_ref, w1, b1, w2, b2, wo, bo, out_ref):
  g = g_ref[...].astype(_F32)
  nrm = jnp.sqrt(jnp.sum(g * g, axis=1, keepdims=True))
  g = g / jnp.maximum(nrm, 1e-12)
  pooled = jnp.mean(g.reshape(ncry, apc, g.shape[1]), axis=1)
  h = jax.nn.softplus(
      jnp.dot(pooled, w1[...], preferred_element_type=_F32) + b1[...])
  h = jax.nn.softplus(
      jnp.dot(h, w2[...], preferred_element_type=_F32) + b2[...])
  out_ref[...] = jnp.dot(h, wo[...], preferred_element_type=_F32) + bo[...]


def _block_diag2(w):
  af_dim = w.shape[0]
  z = jnp.zeros((af_dim, af_dim), _F32)
  return jnp.concatenate([
      jnp.concatenate([w, z], axis=1),
      jnp.concatenate([z, w], axis=1),
  ], axis=0)


def _w8_expand(we, nbr_dim, af_dim):
  w8 = jnp.zeros((8 * nbr_dim, 8 * af_dim), _F32)
  for j in range(4):
    for h in range(2):
      w8 = lax.dynamic_update_slice(
          w8, we, (32 * j + nbr_dim * h, 128 * j + af_dim * h))
  return w8


def _forward(atom_fea, nbr_fea, nbr_fea_idx, crystal_atom_idx, mask, w_emb,
             conv_params, fc1_w, fc1_b, fc2_w, fc2_b, out_w, out_b,
             gather_fn):
  n, orig = atom_fea.shape
  m = nbr_fea.shape[1]
  nbr_dim = nbr_fea.shape[2]
  af_dim = w_emb.shape[0]
  ncry, apc = crystal_atom_idx.shape

  r = _pick_block(n // 2, 1000)
  blocks = n // r
  quarter = r * m // 8

  idx_flat = nbr_fea_idx.astype(jnp.int32).reshape(-1)
  cidx_flat = crystal_atom_idx.astype(jnp.int32).reshape(-1)
  nbrp = nbr_fea.reshape(blocks, 4, quarter, 2 * nbr_dim).transpose(
      0, 2, 1, 3).reshape(blocks * quarter, 8 * nbr_dim)

  masked, af = _embed(atom_fea, mask.reshape(1, orig), w_emb.T)

  eye = jnp.eye(af_dim, dtype=_F32)
  p_mat = jnp.concatenate([eye, eye], axis=0)
  pt_mat = jnp.concatenate([eye, eye], axis=1)

  for (fw, fb, g1, b1, g2, b2) in conv_params:
    wsf = fw[0:af_dim, 0:af_dim].T
    wsc = fw[af_dim:2 * af_dim, 0:af_dim].T
    wnf = fw[0:af_dim, af_dim:2 * af_dim].T
    wnc = fw[af_dim:2 * af_dim, af_dim:2 * af_dim].T
    wef = fw[0:af_dim, 2 * af_dim:].T
    wec = fw[af_dim:2 * af_dim, 2 * af_dim:].T
    wbdf = _block_diag2(wnf)
    wbdc = _block_diag2(wnc)
    w8f = _w8_expand(wef, nbr_dim, af_dim)
    w8c = _w8_expand(wec, nbr_dim, af_dim)
    wspf = jnp.concatenate([wsf, wsf], axis=1)
    wspc = jnp.concatenate([wsc, wsc], axis=1)
    bpf = jnp.tile(fb[0:af_dim].reshape(1, af_dim), (1, 2))
    bpc = jnp.tile(fb[af_dim:].reshape(1, af_dim), (1, 2))
    g1f = g1[0:af_dim].reshape(1, af_dim)
    g1c = g1[af_dim:].reshape(1, af_dim)
    b1f = b1[0:af_dim].reshape(1, af_dim)
    b1c = b1[af_dim:].reshape(1, af_dim)
    gpk_halves = [
        gather_fn(af, idx_flat[h * (n * m // 2):(h + 1) * (n * m // 2)]
                  ).reshape(n * m // 4, 2 * af_dim)
        for h in (0, 1)
    ]
    af_halves = _conv_layer(
        [af[:n // 2], af[n // 2:]], gpk_halves,
        [nbrp[:nbrp.shape[0] // 2], nbrp[nbrp.shape[0] // 2:]], n, m, r,
        wbdf, w8f, wspf, bpf, wbdc, w8c, wspc, bpc, g1f, b1f, g1c, b1c,
        g2.reshape(1, af_dim), b2.reshape(1, af_dim), p_mat, pt_mat)
    af = jnp.concatenate(af_halves, axis=0)

  g_cry = gather_fn(af, cidx_flat).reshape(ncry * apc, af_dim)

  wo_pad = jnp.pad(out_w.T, ((0, 0), (0, 128 - out_w.shape[0])))
  bo_pad = jnp.pad(out_b.reshape(1, -1), ((0, 0), (0, 128 - out_b.shape[0])))
  out = pl.pallas_call(
      functools.partial(_readout_body, ncry, apc),
      out_shape=jax.ShapeDtypeStruct((ncry, 128), _F32),
  )(g_cry, fc1_w.T, fc1_b.reshape(1, af_dim), fc2_w.T,
    fc2_b.reshape(1, af_dim), wo_pad, bo_pad)
  props = out[:, 0:1]
  return props, masked


def kernel(atom_fea, nbr_fea, nbr_fea_idx, crystal_atom_idx, mask, w_emb,
           conv0_fc_w, conv0_fc_b, conv0_bn1_g, conv0_bn1_b, conv0_bn2_g,
           conv0_bn2_b, conv1_fc_w, conv1_fc_b, conv1_bn1_g, conv1_bn1_b,
           conv1_bn2_g, conv1_bn2_b, conv2_fc_w, conv2_fc_b, conv2_bn1_g,
           conv2_bn1_b, conv2_bn2_g, conv2_bn2_b, fc1_w, fc1_b, fc2_w, fc2_b,
           out_w, out_b):
  conv_params = [
      (conv0_fc_w, conv0_fc_b, conv0_bn1_g, conv0_bn1_b, conv0_bn2_g,
       conv0_bn2_b),
      (conv1_fc_w, conv1_fc_b, conv1_bn1_g, conv1_bn1_b, conv1_bn2_g,
       conv1_bn2_b),
      (conv2_fc_w, conv2_fc_b, conv2_bn1_g, conv2_bn1_b, conv2_bn2_g,
       conv2_bn2_b),
  ]
  return _forward(atom_fea, nbr_fea, nbr_fea_idx, crystal_atom_idx, mask,
                  w_emb, conv_params, fc1_w, fc1_b, fc2_w, fc2_b, out_w,
                  out_b, _sc_gather)


# speedup vs baseline: 1.0732x; 1.0732x over previous
import functools

import jax
import jax.numpy as jnp
from jax import lax
from jax.experimental import pallas as pl
from jax.experimental.pallas import tpu as pltpu
from jax.experimental.pallas import tpu_sc as plsc

_F32 = jnp.float32


def _pick_k(n_chunks):
  for k in (10, 8, 5, 4, 2, 1):
    if n_chunks % k == 0 and n_chunks // k >= 32:
      return k
  for k in (10, 8, 5, 4, 2, 1):
    if n_chunks % k == 0:
      return k
  return 1


def _sc_gather(table, idx_flat):
  v, d = table.shape
  dt = table.dtype
  b = idx_flat.shape[0]
  n_chunks = b // 128
  k = _pick_k(n_chunks)
  c = k * 128
  n_sup = n_chunks // k
  n_loop = -(-n_sup // 32)
  idx3d = idx_flat.reshape(n_sup, k, 128)
  mesh = plsc.VectorSubcoreMesh(core_axis_name="c", subcore_axis_name="s")

  @functools.partial(
      pl.kernel,
      mesh=mesh,
      compiler_params=pltpu.CompilerParams(use_tc_tiling_on_sc=False),
      out_type=jax.ShapeDtypeStruct((b, d), dt),
      scratch_types=[
          pltpu.VMEM((k, 128), jnp.int32),
          pltpu.VMEM((c, d), dt),
          pltpu.SemaphoreType.DMA,
      ],
  )
  def gather(table_hbm, idx_hbm, out_hbm, idx_v, rows_v, sem):
    wid = lax.axis_index("s") * 2 + lax.axis_index("c")

    def body(s, carry):
      sup = s * 32 + wid

      @pl.when(sup < n_sup)
      def _():
        pltpu.sync_copy(idx_hbm.at[sup], idx_v)
        copies = [
            pltpu.async_copy(
                table_hbm.at[idx_v.at[j]],
                rows_v.at[pl.ds(j * 128, 128)],
                sem,
            )
            for j in range(k)
        ]
        for cp in copies:
          cp.wait()
        pltpu.sync_copy(rows_v, out_hbm.at[pl.ds(sup * c, c)])

      return carry

    lax.fori_loop(0, n_loop, body, 0)

  return gather(table, idx3d)


def _pick_block(n, cap):
  for r in range(min(n, cap), 0, -1):
    if n % r == 0 and (r % 8 == 0 or r == n):
      return r
  return n


def _embed_body(a_ref, m_ref, w_ref, masked_ref, af_ref):
  masked = a_ref[...] * m_ref[...]
  masked_ref[...] = masked
  af_ref[...] = jnp.dot(masked, w_ref[...], preferred_element_type=_F32)


def _embed(atom_fea, mask2d, w_embt):
  n, orig = atom_fea.shape
  af_dim = w_embt.shape[1]
  r = _pick_block(n, 2000)
  grid = (n // r,)
  return pl.pallas_call(
      _embed_body,
      grid=grid,
      in_specs=[
          pl.BlockSpec((r, orig), lambda i: (i, 0)),
          pl.BlockSpec((1, orig), lambda i: (0, 0)),
          pl.BlockSpec((orig, af_dim), lambda i: (0, 0)),
      ],
      out_specs=[
          pl.BlockSpec((r, orig), lambda i: (i, 0)),
          pl.BlockSpec((r, af_dim), lambda i: (i, 0)),
      ],
      out_shape=[
          jax.ShapeDtypeStruct((n, orig), _F32),
          jax.ShapeDtypeStruct((n, af_dim), _F32),
      ],
  )(atom_fea, mask2d, w_embt)


def _edge_proj(gpk_ref, nbrp_ref, af_ref, wbd, w8, wsp, bp):
  ep = jnp.dot(gpk_ref[...], wbd, preferred_element_type=_F32)
  nq = jnp.dot(nbrp_ref[...], w8, preferred_element_type=_F32)
  parts = [nq[:, 128 * j:128 * (j + 1)] for j in range(4)]
  ep = ep + jnp.concatenate(parts, axis=0)
  sp = jnp.dot(af_ref[...], wsp, preferred_element_type=_F32)
  sp = sp + bp
  return ep, sp


def _stats_half(r, ppa, e, s):
  esum = jnp.sum(e.reshape(r, ppa, 128), axis=1)
  tot = jnp.sum(esum, axis=0) + ppa * jnp.sum(s, axis=0)
  tot2 = (jnp.sum(e * e, axis=0) + 2.0 * jnp.sum(s * esum, axis=0)
          + ppa * jnp.sum(s * s, axis=0))
  return tot, tot2


def _stats_body(m, gpk_ref, nbrp_ref, af_ref, wbdf, w8f, wspf, bpf, wbdc, w8c,
                wspc, bpc, out_ref):
  r = af_ref.shape[0]
  ppa = m // 2
  ef, sf = _edge_proj(gpk_ref, nbrp_ref, af_ref, wbdf[...], w8f[...],
                      wspf[...], bpf[...])
  ec, sc2 = _edge_proj(gpk_ref, nbrp_ref, af_ref, wbdc[...], w8c[...],
                       wspc[...], bpc[...])
  tf, tf2 = _stats_half(r, ppa, ef, sf)
  tc, tc2 = _stats_half(r, ppa, ec, sc2)

  @pl.when(pl.program_id(0) == 0)
  def _():
    out_ref[...] = jnp.zeros_like(out_ref)

  out_ref[0:1, :] += tf[None, :]
  out_ref[1:2, :] += tf2[None, :]
  out_ref[2:3, :] += tc[None, :]
  out_ref[3:4, :] += tc2[None, :]


def _main_body(m, cnt, gpk_ref, nbrp_ref, af_ref, wbdf, w8f, wspf, bpf, wbdc,
               w8c, wspc, bpc, st_ref, g1f, b1f, g1c, b1c, p_ref, pt_ref,
               sum_ref, out2_ref):
  r = af_ref.shape[0]
  pairs_per_atom = m // 2
  inv = 1.0 / cnt
  p = p_ref[...]
  pt = pt_ref[...]
  st = jnp.dot(st_ref[...], p, preferred_element_type=_F32)
  mf = st[0:1, :] * inv
  vf = st[1:2, :] * inv - mf * mf
  scf = g1f[...] * lax.rsqrt(vf + 1e-5)
  shf = b1f[...] - mf * scf
  mc = st[2:3, :] * inv
  vc = st[3:4, :] * inv - mc * mc
  scc = g1c[...] * lax.rsqrt(vc + 1e-5)
  shc = b1c[...] - mc * scc
  scf_p = jnp.dot(scf, pt, preferred_element_type=_F32)
  shf_p = jnp.dot(shf, pt, preferred_element_type=_F32)
  scc_p = jnp.dot(scc, pt, preferred_element_type=_F32)
  shc_p = jnp.dot(shc, pt, preferred_element_type=_F32)

  scf_p4 = jnp.concatenate([scf_p] * 4, axis=1)
  scc_p4 = jnp.concatenate([scc_p] * 4, axis=1)
  ef, sf = _edge_proj(gpk_ref, nbrp_ref, af_ref, wbdf[...] * scf_p,
                      w8f[...] * scf_p4, wspf[...] * scf_p,
                      bpf[...] * scf_p + shf_p)
  ec, sc2 = _edge_proj(gpk_ref, nbrp_ref, af_ref, wbdc[...] * scc_p,
                       w8c[...] * scc_p4, wspc[...] * scc_p,
                       bpc[...] * scc_p + shc_p)

  filt = jax.nn.sigmoid(ef.reshape(r, pairs_per_atom, 128) + sf[:, None, :])
  core = jax.nn.softplus(ec.reshape(r, pairs_per_atom, 128) + sc2[:, None, :])
  psum = jnp.sum(filt * core, axis=1)
  sm = jnp.dot(psum, p, preferred_element_type=_F32)
  sum_ref[...] = sm

  @pl.when(pl.program_id(0) == 0)
  def _():
    out2_ref[...] = jnp.zeros_like(out2_ref)

  out2_ref[0:1, :] += jnp.sum(sm, axis=0)[None, :]
  out2_ref[1:2, :] += jnp.sum(sm * sm, axis=0)[None, :]


def _update_body(cnt, af_ref, sm_ref, st_ref, g2, b2, out_ref):
  inv = 1.0 / cnt
  mu = st_ref[0:1, :] * inv
  var = st_ref[1:2, :] * inv - mu * mu
  s = g2[...] * lax.rsqrt(var + 1e-5)
  sh = b2[...] - mu * s
  out_ref[...] = jax.nn.softplus(af_ref[...] + sm_ref[...] * s + sh)


def _conv_layer(af, gpk, nbrp, m, r, wbdf, w8f, wspf, bpf, wbdc, w8c, wspc,
                bpc, g1f, b1f, g1c, b1c, g2, b2, p_mat, pt_mat):
  n, af_dim = af.shape
  grid = (n // r,)
  pack_r = r * m // 2
  nbr_r = nbrp.shape[0] // (n // r)

  g_spec = pl.BlockSpec((pack_r, 128), lambda i: (i, 0))
  nbr_spec = pl.BlockSpec((nbr_r, 128), lambda i: (i, 0))
  af_spec = pl.BlockSpec((r, af_dim), lambda i: (i, 0))
  wbd_spec = pl.BlockSpec((128, 128), lambda i: (0, 0))
  w8_spec = pl.BlockSpec((128, 512), lambda i: (0, 0))
  wsp_spec = pl.BlockSpec((af_dim, 128), lambda i: (0, 0))
  bp_spec = pl.BlockSpec((1, 128), lambda i: (0, 0))
  b_spec = pl.BlockSpec((1, af_dim), lambda i: (0, 0))
  stp_spec = pl.BlockSpec((8, 128), lambda i: (0, 0))
  st_spec = pl.BlockSpec((8, af_dim), lambda i: (0, 0))
  p_spec = pl.BlockSpec((128, af_dim), lambda i: (0, 0))
  pt_spec = pl.BlockSpec((af_dim, 128), lambda i: (0, 0))

  stats = pl.pallas_call(
      functools.partial(_stats_body, m),
      grid=grid,
      in_specs=[g_spec, nbr_spec, af_spec, wbd_spec, w8_spec, wsp_spec,
                bp_spec, wbd_spec, w8_spec, wsp_spec, bp_spec],
      out_specs=stp_spec,
      out_shape=jax.ShapeDtypeStruct((8, 128), _F32),
  )(gpk, nbrp, af, wbdf, w8f, wspf, bpf, wbdc, w8c, wspc, bpc)

  summed, st2 = pl.pallas_call(
      functools.partial(_main_body, m, float(n * m)),
      grid=grid,
      in_specs=[g_spec, nbr_spec, af_spec, wbd_spec, w8_spec, wsp_spec,
                bp_spec, wbd_spec, w8_spec, wsp_spec, bp_spec, stp_spec,
                b_spec, b_spec, b_spec, b_spec, p_spec, pt_spec],
      out_specs=[af_spec, st_spec],
      out_shape=[
          jax.ShapeDtypeStruct((n, af_dim), _F32),
          jax.ShapeDtypeStruct((8, af_dim), _F32),
      ],
  )(gpk, nbrp, af, wbdf, w8f, wspf, bpf, wbdc, w8c, wspc, bpc, stats,
    g1f, b1f, g1c, b1c, p_mat, pt_mat)

  r2 = _pick_block(n, 2000)
  return pl.pallas_call(
      functools.partial(_update_body, float(n)),
      grid=(n // r2,),
      in_specs=[
          pl.BlockSpec((r2, af_dim), lambda i: (i, 0)),
          pl.BlockSpec((r2, af_dim), lambda i: (i, 0)),
          st_spec,
          b_spec,
          b_spec,
      ],
      out_specs=pl.BlockSpec((r2, af_dim), lambda i: (i, 0)),
      out_shape=jax.ShapeDtypeStruct((n, af_dim), _F32),
  )(af, summed, st2, g2, b2)


def _readout_body(ncry, apc, g_ref, w1, b1, w2, b2, wo, bo, out_ref):
  g = g_ref[...].astype(_F32)
  nrm = jnp.sqrt(jnp.sum(g * g, axis=1, keepdims=True))
  g = g / jnp.maximum(nrm, 1e-12)
  pooled = jnp.mean(g.reshape(ncry, apc, g.shape[1]), axis=1)
  h = jax.nn.softplus(
      jnp.dot(pooled, w1[...], preferred_element_type=_F32) + b1[...])
  h = jax.nn.softplus(
      jnp.dot(h, w2[...], preferred_element_type=_F32) + b2[...])
  out_ref[...] = jnp.dot(h, wo[...], preferred_element_type=_F32) + bo[...]


def _block_diag2(w):
  af_dim = w.shape[0]
  z = jnp.zeros((af_dim, af_dim), _F32)
  return jnp.concatenate([
      jnp.concatenate([w, z], axis=1),
      jnp.concatenate([z, w], axis=1),
  ], axis=0)


def _w8_expand(we, nbr_dim, af_dim):
  w8 = jnp.zeros((8 * nbr_dim, 8 * af_dim), _F32)
  for j in range(4):
    for h in range(2):
      w8 = lax.dynamic_update_slice(
          w8, we, (32 * j + nbr_dim * h, 128 * j + af_dim * h))
  return w8


def _forward(atom_fea, nbr_fea, nbr_fea_idx, crystal_atom_idx, mask, w_emb,
             conv_params, fc1_w, fc1_b, fc2_w, fc2_b, out_w, out_b,
             gather_fn):
  n, orig = atom_fea.shape
  m = nbr_fea.shape[1]
  nbr_dim = nbr_fea.shape[2]
  af_dim = w_emb.shape[0]
  ncry, apc = crystal_atom_idx.shape

  r = _pick_block(n, 1000)
  blocks = n // r
  quarter = r * m // 8

  idx_flat = nbr_fea_idx.astype(jnp.int32).reshape(-1)
  cidx_flat = crystal_atom_idx.astype(jnp.int32).reshape(-1)
  nbrp = nbr_fea.reshape(blocks, 4, quarter, 2 * nbr_dim).transpose(
      0, 2, 1, 3).reshape(blocks * quarter, 8 * nbr_dim)

  masked, af = _embed(atom_fea, mask.reshape(1, orig), w_emb.T)

  eye = jnp.eye(af_dim, dtype=_F32)
  p_mat = jnp.concatenate([eye, eye], axis=0)
  pt_mat = jnp.concatenate([eye, eye], axis=1)

  for (fw, fb, g1, b1, g2, b2) in conv_params:
    wsf = fw[0:af_dim, 0:af_dim].T
    wsc = fw[af_dim:2 * af_dim, 0:af_dim].T
    wnf = fw[0:af_dim, af_dim:2 * af_dim].T
    wnc = fw[af_dim:2 * af_dim, af_dim:2 * af_dim].T
    wef = fw[0:af_dim, 2 * af_dim:].T
    wec = fw[af_dim:2 * af_dim, 2 * af_dim:].T
    wbdf = _block_diag2(wnf)
    wbdc = _block_diag2(wnc)
    w8f = _w8_expand(wef, nbr_dim, af_dim)
    w8c = _w8_expand(wec, nbr_dim, af_dim)
    wspf = jnp.concatenate([wsf, wsf], axis=1)
    wspc = jnp.concatenate([wsc, wsc], axis=1)
    bpf = jnp.tile(fb[0:af_dim].reshape(1, af_dim), (1, 2))
    bpc = jnp.tile(fb[af_dim:].reshape(1, af_dim), (1, 2))
    g1f = g1[0:af_dim].reshape(1, af_dim)
    g1c = g1[af_dim:].reshape(1, af_dim)
    b1f = b1[0:af_dim].reshape(1, af_dim)
    b1c = b1[af_dim:].reshape(1, af_dim)
    gpk = gather_fn(af, idx_flat).reshape(n * m // 2, 2 * af_dim)
    af = _conv_layer(af, gpk, nbrp, m, r, wbdf, w8f, wspf, bpf, wbdc, w8c,
                     wspc, bpc, g1f, b1f, g1c, b1c, g2.reshape(1, af_dim),
                     b2.reshape(1, af_dim), p_mat, pt_mat)

  g_cry = gather_fn(af, cidx_flat).reshape(ncry * apc, af_dim)

  wo_pad = jnp.pad(out_w.T, ((0, 0), (0, 128 - out_w.shape[0])))
  bo_pad = jnp.pad(out_b.reshape(1, -1), ((0, 0), (0, 128 - out_b.shape[0])))
  out = pl.pallas_call(
      functools.partial(_readout_body, ncry, apc),
      out_shape=jax.ShapeDtypeStruct((ncry, 128), _F32),
  )(g_cry, fc1_w.T, fc1_b.reshape(1, af_dim), fc2_w.T,
    fc2_b.reshape(1, af_dim), wo_pad, bo_pad)
  props = out[:, 0:1]
  return props, masked


def kernel(atom_fea, nbr_fea, nbr_fea_idx, crystal_atom_idx, mask, w_emb,
           conv0_fc_w, conv0_fc_b, conv0_bn1_g, conv0_bn1_b, conv0_bn2_g,
           conv0_bn2_b, conv1_fc_w, conv1_fc_b, conv1_bn1_g, conv1_bn1_b,
           conv1_bn2_g, conv1_bn2_b, conv2_fc_w, conv2_fc_b, conv2_bn1_g,
           conv2_bn1_b, conv2_bn2_g, conv2_bn2_b, fc1_w, fc1_b, fc2_w, fc2_b,
           out_w, out_b):
  conv_params = [
      (conv0_fc_w, conv0_fc_b, conv0_bn1_g, conv0_bn1_b, conv0_bn2_g,
       conv0_bn2_b),
      (conv1_fc_w, conv1_fc_b, conv1_bn1_g, conv1_bn1_b, conv1_bn2_g,
       conv1_bn2_b),
      (conv2_fc_w, conv2_fc_b, conv2_bn1_g, conv2_bn1_b, conv2_bn2_g,
       conv2_bn2_b),
  ]
  return _forward(atom_fea, nbr_fea, nbr_fea_idx, crystal_atom_idx, mask,
                  w_emb, conv_params, fc1_w, fc1_b, fc2_w, fc2_b, out_w,
                  out_b, _sc_gather)
